# baseline (speedup 1.0000x reference)
import functools

import jax
import jax.numpy as jnp
from jax import lax
from jax.experimental import pallas as pl
from jax.experimental.pallas import tpu as pltpu
from jax.experimental.pallas import tpu_sc as plsc

TILE_B = 2048


def _argmax_body(xt_ref, idx_ref):
    xt = xt_ref[:]
    in_dim = xt.shape[0]
    iota = lax.broadcasted_iota(jnp.int32, xt.shape, 0)
    m = jnp.max(xt, axis=0, keepdims=True)
    idx_ref[:] = jnp.min(jnp.where(xt == m, iota, in_dim), axis=0)


def _mlp_body(e_ref, w1_ref, b1_ref, w2_ref, b2_ref, wfc_ref, _buf_ref,
              out_ref):
    e = e_ref[:]
    ht = lax.dot_general(w1_ref[:], e, (((1,), (1,)), ((), ())),
                         preferred_element_type=jnp.float32)
    h = jnp.maximum(ht + b1_ref[:], 0.0)
    f = jnp.dot(w2_ref[:], h, preferred_element_type=jnp.float32) + b2_ref[:]
    logits = jnp.dot(wfc_ref[:], f, preferred_element_type=jnp.float32)
    lm = jnp.max(logits, axis=0, keepdims=True)
    p = jnp.exp(logits - lm)
    out_ref[:] = p / jnp.sum(p, axis=0, keepdims=True)


def _make_sc_gather(batch, emb_dim):
    info = plsc.get_sparse_core_info()
    nc, ns = info.num_cores, info.num_subcores
    nw = nc * ns
    b_per_w = batch // nw
    n_chunks = b_per_w // 128
    mesh = plsc.VectorSubcoreMesh(core_axis_name="c", subcore_axis_name="s")

    @functools.partial(
        pl.kernel,
        out_type=jax.ShapeDtypeStruct((batch, emb_dim), jnp.float32),
        mesh=mesh,
        compiler_params=pltpu.CompilerParams(use_tc_tiling_on_sc=False),
        scratch_types=[
            pltpu.VMEM((b_per_w,), jnp.int32),
            pltpu.VMEM((b_per_w, emb_dim), jnp.float32),
            pltpu.SemaphoreType.DMA,
        ],
    )
    def gather(emb_hbm, idx_hbm, out_hbm, idx_v, rows_v, sem):
        wid = lax.axis_index("s") * nc + lax.axis_index("c")
        base = wid * b_per_w
        pltpu.sync_copy(idx_hbm.at[pl.ds(base, b_per_w)], idx_v)
        copies = [
            pltpu.async_copy(
                emb_hbm.at[idx_v.at[pl.ds(j * 128, 128)]],
                rows_v.at[pl.ds(j * 128, 128)],
                sem,
            )
            for j in range(n_chunks)
        ]
        for c in copies:
            c.wait()
        pltpu.sync_copy(rows_v, out_hbm.at[pl.ds(base, b_per_w)])

    return gather


N_CHUNKS = 2


@jax.jit
def kernel(x, emb, W1, b1, W2, b2, Wfc):
    batch, in_dim = x.shape
    emb_dim = emb.shape[1]
    hid = W1.shape[0]
    out_dim = Wfc.shape[0]
    cb = batch // N_CHUNKS
    grid = cb // TILE_B

    xt = x.T
    b1c = b1.reshape(hid, 1)
    b2c = b2.reshape(hid, 1)

    sc_gather = _make_sc_gather(cb, emb_dim)
    full = lambda shape: pl.BlockSpec(shape, lambda i: (0, 0))

    idxs = [
        pl.pallas_call(
            _argmax_body,
            grid=(grid,),
            in_specs=[
                pl.BlockSpec((in_dim, TILE_B), lambda i, c=c: (0, c * grid + i))
            ],
            out_specs=pl.BlockSpec((TILE_B,), lambda i: (i,)),
            out_shape=jax.ShapeDtypeStruct((cb,), jnp.int32),
        )(xt)
        for c in range(N_CHUNKS)
    ]
    es = [sc_gather(emb, idx) for idx in idxs]

    outt = None
    for c, e in enumerate(es):
        args = [e, W1, b1c, W2, b2c, Wfc]
        in_specs = [
            pl.BlockSpec((TILE_B, emb_dim), lambda i: (i, 0)),
            full(W1.shape),
            full(b1c.shape),
            full(W2.shape),
            full(b2c.shape),
            full(Wfc.shape),
        ]
        kwargs = {}
        if c > 0:
            args.append(outt)
            in_specs.append(pl.BlockSpec(memory_space=pl.ANY))
            kwargs["input_output_aliases"] = {len(args) - 1: 0}
            body = _mlp_body
        else:
            body = lambda *refs: _mlp_body(*refs[:-1], None, refs[-1])
        outt = pl.pallas_call(
            body,
            grid=(grid,),
            in_specs=in_specs,
            out_specs=pl.BlockSpec((out_dim, TILE_B),
                                   lambda i, c=c: (0, c * grid + i)),
            out_shape=jax.ShapeDtypeStruct((out_dim, batch), jnp.float32),
            **kwargs,
        )(*args)
    return outt.T


# speedup vs baseline: 1.5570x; 1.5570x over previous
import jax
import jax.numpy as jnp
from jax import lax
from jax.experimental import pallas as pl

TILE_B = 2048


def _fused_body(xt_ref, embt_ref, w1_ref, b1_ref, w2_ref, b2_ref, wfc_ref,
                out_ref):
    xt = xt_ref[:]
    in_dim = xt.shape[0]
    iota = lax.broadcasted_iota(jnp.int32, xt.shape, 0)
    m = jnp.max(xt, axis=0, keepdims=True)
    idx = jnp.min(jnp.where(xt == m, iota, in_dim), axis=0, keepdims=True)
    oh = (iota == idx).astype(jnp.float32)
    e = jnp.dot(embt_ref[:], oh, preferred_element_type=jnp.float32)
    h = jnp.maximum(
        jnp.dot(w1_ref[:], e, preferred_element_type=jnp.float32)
        + b1_ref[:], 0.0)
    f = jnp.dot(w2_ref[:], h, preferred_element_type=jnp.float32) + b2_ref[:]
    logits = jnp.dot(wfc_ref[:], f, preferred_element_type=jnp.float32)
    lm = jnp.max(logits, axis=0, keepdims=True)
    p = jnp.exp(logits - lm)
    out_ref[:] = p / jnp.sum(p, axis=0, keepdims=True)


@jax.jit
def kernel(x, emb, W1, b1, W2, b2, Wfc):
    batch, in_dim = x.shape
    hid = W1.shape[0]
    out_dim = Wfc.shape[0]
    grid = batch // TILE_B

    xt = x.T
    embt = emb.T
    b1c = b1.reshape(hid, 1)
    b2c = b2.reshape(hid, 1)

    full = lambda shape: pl.BlockSpec(shape, lambda i: (0, 0))
    outt = pl.pallas_call(
        _fused_body,
        grid=(grid,),
        in_specs=[
            pl.BlockSpec((in_dim, TILE_B), lambda i: (0, i)),
            full(embt.shape),
            full(W1.shape),
            full(b1c.shape),
            full(W2.shape),
            full(b2c.shape),
            full(Wfc.shape),
        ],
        out_specs=pl.BlockSpec((out_dim, TILE_B), lambda i: (0, i)),
        out_shape=jax.ShapeDtypeStruct((out_dim, batch), jnp.float32),
    )(xt, embt, W1, b1c, W2, b2c, Wfc)
    return outt.T
